# baseline (speedup 1.0000x reference)
import functools

import jax
import jax.numpy as jnp
from jax import lax
from jax.experimental import pallas as pl
from jax.experimental.pallas import tpu as pltpu
from jax.experimental.pallas import tpu_sc as plsc

BATCH = 16384
FEAT = 640
NCLASS = 6
CPAD = 8


SC_ROWS = 7168
NC = 2
NS = 16
NW = NC * NS
RPW = SC_ROWS // NW
CH = 56
NCH = RPW // CH
NVR = FEAT // 16
NACC = 4

_sc_mesh = plsc.VectorSubcoreMesh(core_axis_name="c", subcore_axis_name="s")


@functools.partial(
    pl.kernel,
    out_type=jax.ShapeDtypeStruct((NW, 32), jnp.float32),
    mesh=_sc_mesh,
    scratch_types=[
        pltpu.VMEM((CH, FEAT), jnp.float32),
        pltpu.VMEM((CH, FEAT), jnp.float32),
        pltpu.VMEM((NCLASS * FEAT,), jnp.float32),
        pltpu.VMEM((RPW,), jnp.int32),
        pltpu.VMEM((32,), jnp.float32),
        pltpu.SMEM((RPW,), jnp.int32),
        pltpu.SemaphoreType.DMA,
        pltpu.SemaphoreType.DMA,
    ],
)
def _sc_loss(f_hbm, lab_hbm, c_hbm, out_hbm,
             fbuf0, fbuf1, cbuf, labbuf, obuf, offs, sem0, sem1):
    wid = lax.axis_index("s") * NC + lax.axis_index("c")
    rbase = wid * RPW

    pltpu.sync_copy(c_hbm, cbuf)
    pltpu.sync_copy(lab_hbm.at[pl.ds(rbase, RPW)], labbuf)

    bufs = (fbuf0, fbuf1)
    sems = (sem0, sem1)
    pltpu.async_copy(f_hbm.at[pl.ds(rbase, CH), :], fbuf0, sem0)
    pltpu.async_copy(f_hbm.at[pl.ds(rbase + CH, CH), :], fbuf1, sem1)

    iota = lax.iota(jnp.int32, 16)
    one = jnp.float32(1.0)
    zero = jnp.float32(0.0)

    def _off_body(g, hist):
        labv = labbuf[pl.ds(g * 16, 16)]
        coffv = jnp.minimum(labv, NCLASS - 1) * FEAT
        for t in range(16):
            offs[g * 16 + t] = coffv[t]
            hist = hist + jnp.where(iota == labv[t], one, zero)
        return hist
    hist = lax.fori_loop(0, RPW // 16, _off_body,
                         jnp.zeros((16,), jnp.float32))

    def _compute_chunk(buf, ch, chains):
        def _row_body(r, ra):
            coff = offs[ch * CH + r]
            ra = list(ra)
            for j in range(NVR):
                d = (buf[r, pl.ds(j * 16, 16)]
                     - cbuf[pl.ds(coff + j * 16, 16)])
                ra[j % NACC] = ra[j % NACC] + d * d
            return tuple(ra)

        return lax.fori_loop(0, CH, _row_body, chains)

    def _pair_body(p, chains):
        for b in range(2):
            ch = 2 * p + b
            pltpu.make_async_copy(
                f_hbm.at[pl.ds(0, CH), :], bufs[b], sems[b]).wait()
            chains = _compute_chunk(bufs[b], ch, chains)

            @pl.when(ch + 2 < NCH)
            def _():
                pltpu.async_copy(
                    f_hbm.at[pl.ds(rbase + (ch + 2) * CH, CH), :],
                    bufs[b], sems[b])
        return chains

    chains = lax.fori_loop(
        0, NCH // 2, _pair_body,
        tuple(jnp.zeros((16,), jnp.float32) for _ in range(NACC)))
    f2v = (chains[0] + chains[1]) + (chains[2] + chains[3])

    cntv = jnp.where(iota < NCLASS, hist, zero)
    obuf[pl.ds(0, 16)] = f2v
    obuf[pl.ds(16, 16)] = cntv
    pltpu.sync_copy(obuf, out_hbm.at[wid])


BB = 2048
TCOFF = SC_ROWS // BB
NB = (BATCH - SC_ROWS) // BB


def _tc_body(f_ref, lab_ref, ct_ref, out_ref, acc_ref):
    i = pl.program_id(0)

    @pl.when(i == 0)
    def _():
        acc_ref[0] = 0.0
        acc_ref[1] = 0.0

    f = f_ref[...]
    lab = lab_ref[...]
    ct = ct_ref[...]

    mask = (lab < NCLASS).astype(jnp.float32)
    onehot = (lab == lax.broadcasted_iota(jnp.int32, (BB, CPAD), 1))
    onehot = onehot.astype(jnp.float32) * mask

    p = jnp.dot(f, ct, preferred_element_type=jnp.float32)
    c2 = jnp.sum(ct * ct, axis=0, keepdims=True)
    rows2 = jnp.sum(f * f, axis=1, keepdims=True)

    contrib = jnp.sum(rows2 * mask) + jnp.sum(onehot * (c2 - 2.0 * p))
    acc_ref[0] += contrib
    acc_ref[1] += jnp.sum(mask)

    @pl.when(i == NB - 1)
    def _():
        out_ref[0, 0] = acc_ref[0]
        out_ref[0, 1] = acc_ref[1]


@jax.jit
def _center_loss(features, labels, centers, centers_t):
    sc_part = _sc_loss(features, labels, centers.reshape(-1))
    lab2d = labels.reshape(BATCH, 1)
    tc_out = pl.pallas_call(
        _tc_body,
        grid=(NB,),
        in_specs=[
            pl.BlockSpec((BB, FEAT), lambda i: (i + TCOFF, 0)),
            pl.BlockSpec((BB, 1), lambda i: (i + TCOFF, 0)),
            pl.BlockSpec((FEAT, CPAD), lambda i: (0, 0)),
        ],
        out_specs=pl.BlockSpec(memory_space=pltpu.SMEM),
        out_shape=jax.ShapeDtypeStruct((1, 2), jnp.float32),
        scratch_shapes=[pltpu.SMEM((2,), jnp.float32)],
    )(features, lab2d, centers_t)
    num = tc_out[0, 0] + jnp.sum(sc_part[:, :16])
    cnt = tc_out[0, 1] + jnp.sum(sc_part[:, 16:])
    return num / cnt


def kernel(features, labels, centers):
    centers_t = jnp.zeros((FEAT, CPAD), jnp.float32).at[:, :NCLASS].set(
        centers.T)
    return _center_loss(features, labels, centers, centers_t)


# speedup vs baseline: 1.0400x; 1.0400x over previous
import functools

import jax
import jax.numpy as jnp
from jax import lax
from jax.experimental import pallas as pl
from jax.experimental.pallas import tpu as pltpu
from jax.experimental.pallas import tpu_sc as plsc

BATCH = 16384
FEAT = 640
NCLASS = 6
CPAD = 8


SC_ROWS = 6144
NC = 2
NS = 16
NW = NC * NS
RPW = SC_ROWS // NW
CH = 48
NCH = RPW // CH
NVR = FEAT // 16
NACC = 4

assert SC_ROWS % (NW * 16) == 0 and RPW % CH == 0 and NCH % 2 == 0

_sc_mesh = plsc.VectorSubcoreMesh(core_axis_name="c", subcore_axis_name="s")


@functools.partial(
    pl.kernel,
    out_type=jax.ShapeDtypeStruct((NW, 32), jnp.float32),
    mesh=_sc_mesh,
    scratch_types=[
        pltpu.VMEM((CH, FEAT), jnp.float32),
        pltpu.VMEM((CH, FEAT), jnp.float32),
        pltpu.VMEM((NCLASS * FEAT,), jnp.float32),
        pltpu.VMEM((RPW,), jnp.int32),
        pltpu.VMEM((32,), jnp.float32),
        pltpu.SMEM((RPW,), jnp.int32),
        pltpu.SemaphoreType.DMA,
        pltpu.SemaphoreType.DMA,
    ],
)
def _sc_loss(f_hbm, lab_hbm, c_hbm, out_hbm,
             fbuf0, fbuf1, cbuf, labbuf, obuf, offs, sem0, sem1):
    wid = lax.axis_index("s") * NC + lax.axis_index("c")
    rbase = wid * RPW

    pltpu.sync_copy(c_hbm, cbuf)
    pltpu.sync_copy(lab_hbm.at[pl.ds(rbase, RPW)], labbuf)

    bufs = (fbuf0, fbuf1)
    sems = (sem0, sem1)
    pltpu.async_copy(f_hbm.at[pl.ds(rbase, CH), :], fbuf0, sem0)
    pltpu.async_copy(f_hbm.at[pl.ds(rbase + CH, CH), :], fbuf1, sem1)

    iota = lax.iota(jnp.int32, 16)
    one = jnp.float32(1.0)
    zero = jnp.float32(0.0)

    def _off_body(g, hist):
        labv = labbuf[pl.ds(g * 16, 16)]
        coffv = jnp.minimum(labv, NCLASS - 1) * FEAT
        for t in range(16):
            offs[g * 16 + t] = coffv[t]
            hist = hist + jnp.where(iota == labv[t], one, zero)
        return hist
    hist = lax.fori_loop(0, RPW // 16, _off_body,
                         jnp.zeros((16,), jnp.float32))

    def _compute_chunk(buf, ch, chains):
        def _row_body(r, ra):
            coff = offs[ch * CH + r]
            ra = list(ra)
            for j in range(NVR):
                d = (buf[r, pl.ds(j * 16, 16)]
                     - cbuf[pl.ds(coff + j * 16, 16)])
                ra[j % NACC] = ra[j % NACC] + d * d
            return tuple(ra)

        return lax.fori_loop(0, CH, _row_body, chains)

    def _pair_body(p, chains):
        for b in range(2):
            ch = 2 * p + b
            pltpu.make_async_copy(
                f_hbm.at[pl.ds(0, CH), :], bufs[b], sems[b]).wait()
            chains = _compute_chunk(bufs[b], ch, chains)

            @pl.when(ch + 2 < NCH)
            def _():
                pltpu.async_copy(
                    f_hbm.at[pl.ds(rbase + (ch + 2) * CH, CH), :],
                    bufs[b], sems[b])
        return chains

    chains = lax.fori_loop(
        0, NCH // 2, _pair_body,
        tuple(jnp.zeros((16,), jnp.float32) for _ in range(NACC)))
    f2v = (chains[0] + chains[1]) + (chains[2] + chains[3])

    cntv = jnp.where(iota < NCLASS, hist, zero)
    obuf[pl.ds(0, 16)] = f2v
    obuf[pl.ds(16, 16)] = cntv
    pltpu.sync_copy(obuf, out_hbm.at[wid])


BB = 2048
TCOFF = SC_ROWS // BB
NB = (BATCH - SC_ROWS) // BB

assert SC_ROWS % BB == 0 and (BATCH - SC_ROWS) % BB == 0


def _tc_body(f_ref, lab_ref, ct_ref, out_ref, acc_ref):
    i = pl.program_id(0)

    @pl.when(i == 0)
    def _():
        acc_ref[0] = 0.0
        acc_ref[1] = 0.0

    f = f_ref[...]
    lab = lab_ref[...]
    ct = ct_ref[...]

    mask = (lab < NCLASS).astype(jnp.float32)
    onehot = (lab == lax.broadcasted_iota(jnp.int32, (BB, CPAD), 1))
    onehot = onehot.astype(jnp.float32) * mask

    p = jnp.dot(f, ct, preferred_element_type=jnp.float32)
    c2 = jnp.sum(ct * ct, axis=0, keepdims=True)
    rows2 = jnp.sum(f * f, axis=1, keepdims=True)

    contrib = jnp.sum(rows2 * mask) + jnp.sum(onehot * (c2 - 2.0 * p))
    acc_ref[0] += contrib
    acc_ref[1] += jnp.sum(mask)

    @pl.when(i == NB - 1)
    def _():
        out_ref[0, 0] = acc_ref[0]
        out_ref[0, 1] = acc_ref[1]


@jax.jit
def _center_loss(features, labels, centers, centers_t):
    sc_part = _sc_loss(features, labels, centers.reshape(-1))
    lab2d = labels.reshape(BATCH, 1)
    tc_out = pl.pallas_call(
        _tc_body,
        grid=(NB,),
        in_specs=[
            pl.BlockSpec((BB, FEAT), lambda i: (i + TCOFF, 0)),
            pl.BlockSpec((BB, 1), lambda i: (i + TCOFF, 0)),
            pl.BlockSpec((FEAT, CPAD), lambda i: (0, 0)),
        ],
        out_specs=pl.BlockSpec(memory_space=pltpu.SMEM),
        out_shape=jax.ShapeDtypeStruct((1, 2), jnp.float32),
        scratch_shapes=[pltpu.SMEM((2,), jnp.float32)],
    )(features, lab2d, centers_t)
    num = tc_out[0, 0] + jnp.sum(sc_part[:, :16])
    cnt = tc_out[0, 1] + jnp.sum(sc_part[:, 16:])
    return num / cnt


def kernel(features, labels, centers):
    centers_t = jnp.zeros((FEAT, CPAD), jnp.float32).at[:, :NCLASS].set(
        centers.T)
    return _center_loss(features, labels, centers, centers_t)
